# baseline (speedup 1.0000x reference)
import jax
import jax.numpy as jnp
from jax import lax
from jax.experimental import pallas as pl

RADIUS2 = 0.1 * 0.1
NS = 32
N = 8192
M = 2048
MB = 256
BIG = 1.0e9


def _kernel(qm_ref, xm_ref, xr_ref, qrep_ref, wb_ref, w1_ref, b1_ref,
            w2_ref, b2_ref, out_ref):
    qm = qm_ref[0]
    xm = xm_ref[0]
    qn = jnp.sum(qm * qm, axis=1)[:, None]
    xn = jnp.sum(xm * xm, axis=1)[None, :]
    cross = lax.dot_general(qm.astype(jnp.bfloat16), xm.astype(jnp.bfloat16),
                            (((1,), (1,)), ((), ())),
                            preferred_element_type=jnp.float32)
    d2 = qn + xn - 2.0 * cross

    CH = 128
    NCH = N // CH
    maskf = (d2 < RADIUS2).astype(jnp.bfloat16).reshape(MB, NCH, CH)
    ti = lax.broadcasted_iota(jnp.int32, (CH, CH), 0)
    tj = lax.broadcasted_iota(jnp.int32, (CH, CH), 1)
    tri = (ti <= tj).astype(jnp.bfloat16)
    si = lax.broadcasted_iota(jnp.int32, (NCH, NCH), 0)
    sj = lax.broadcasted_iota(jnp.int32, (NCH, NCH), 1)
    stri = (si < sj).astype(jnp.bfloat16)
    intra = lax.dot_general(maskf, tri, (((2,), (0,)), ((), ())),
                            preferred_element_type=jnp.float32)
    chunk_tot = intra[:, :, CH - 1]
    chunk_excl = lax.dot_general(chunk_tot.astype(jnp.bfloat16), stri,
                                 (((1,), (0,)), ((), ())),
                                 preferred_element_type=jnp.float32)
    rank = intra - 1.0 + chunk_excl[:, :, None]
    slotv = jnp.where(maskf > 0, rank, -1.0).reshape(MB, N)
    iota_n = lax.broadcasted_iota(jnp.int32, (MB, N), 1).astype(jnp.float32)
    tn = jnp.where(slotv >= 0, iota_n, 0.0)
    total = (chunk_excl[:, NCH - 1] + chunk_tot[:, NCH - 1])[:, None]
    iota32 = lax.broadcasted_iota(jnp.int32, (1, NS), 1)
    idxf = jnp.zeros((MB, NS), jnp.float32)
    for s in range(NS):
        idx_s = jnp.sum(jnp.where(slotv == float(s), tn, 0.0),
                        axis=1, keepdims=True)
        idxf = idxf + idx_s * (iota32 == s).astype(jnp.float32)
    valid = (iota32.astype(jnp.float32)
             < jnp.minimum(total, float(NS))).astype(jnp.float32)

    hi = jnp.floor(idxf * (1.0 / 64.0))
    lo = idxf - hi * 64.0
    ih = lax.broadcasted_iota(jnp.int32, (MB, NS, 128), 2).astype(jnp.float32)
    oh = (hi[:, :, None] == ih).astype(jnp.float32)
    il = lax.broadcasted_iota(jnp.int32, (MB, NS, 64), 2).astype(jnp.float32)
    ol = (lo[:, :, None] == il).astype(jnp.float32)

    g1 = lax.dot_general(oh, xr_ref[0], (((2,), (0,)), ((), ())),
                         precision=lax.Precision.HIGHEST,
                         preferred_element_type=jnp.float32)
    oltile = jnp.concatenate([ol, ol, ol], axis=2)
    g2 = g1 * oltile
    rel192 = g2 - qrep_ref[0][:, None, :] * oltile
    h1p = lax.dot_general(rel192.astype(jnp.bfloat16),
                          wb_ref[...].astype(jnp.bfloat16),
                          (((2,), (0,)), ((), ())),
                          preferred_element_type=jnp.float32)
    h1 = jnp.maximum(h1p + b1_ref[0][None, None, :], 0.0)
    h2p = lax.dot_general(h1.astype(jnp.bfloat16),
                          w2_ref[...].astype(jnp.bfloat16),
                          (((2,), (1,)), ((), ())),
                          preferred_element_type=jnp.float32)
    h2 = jnp.maximum(h2p + b2_ref[0][None, None, :], 0.0)
    pooled = jnp.max(h2 * valid[:, :, None], axis=1)
    out_ref[...] = pooled[None]


def kernel(xyz, new_xyz, W1, b1, W2, b2):
    B = xyz.shape[0]
    xr = xyz.reshape(B, 128, 64, 3).transpose(0, 1, 3, 2).reshape(B, 128, 192)
    qrep = jnp.repeat(new_xyz, 64, axis=2)
    wb = jnp.repeat(W1.T, 64, axis=0)
    b1r = b1.reshape(1, 32)
    b2r = b2.reshape(1, 32)

    grid = (B, M // MB)
    out = pl.pallas_call(
        _kernel,
        grid=grid,
        in_specs=[
            pl.BlockSpec((1, MB, 3), lambda b, m: (b, m, 0)),
            pl.BlockSpec((1, N, 3), lambda b, m: (b, 0, 0)),
            pl.BlockSpec((1, 128, 192), lambda b, m: (b, 0, 0)),
            pl.BlockSpec((1, MB, 192), lambda b, m: (b, m, 0)),
            pl.BlockSpec((192, 32), lambda b, m: (0, 0)),
            pl.BlockSpec((32, 3), lambda b, m: (0, 0)),
            pl.BlockSpec((1, 32), lambda b, m: (0, 0)),
            pl.BlockSpec((32, 32), lambda b, m: (0, 0)),
            pl.BlockSpec((1, 32), lambda b, m: (0, 0)),
        ],
        out_specs=pl.BlockSpec((1, MB, 32), lambda b, m: (b, m, 0)),
        out_shape=jax.ShapeDtypeStruct((B, M, 32), jnp.float32),
    )(new_xyz, xyz, xr, qrep, wb, W1, b1r, W2, b2r)
    feats = out.transpose(0, 2, 1)
    return new_xyz, feats


# speedup vs baseline: 1.2153x; 1.2153x over previous
import jax
import jax.numpy as jnp
from jax import lax
from jax.experimental import pallas as pl

RADIUS2 = 0.1 * 0.1
NS = 32
N = 8192
M = 2048
MB = 256
BIG = 1.0e9


def _kernel(qm_ref, xm_ref, xr_ref, qrep_ref, wb_ref, w1_ref, b1_ref,
            w2_ref, b2_ref, out_ref):
    qm = qm_ref[0]
    xm = xm_ref[0]
    qn = jnp.sum(qm * qm, axis=1)[:, None]
    xn = jnp.sum(xm * xm, axis=1)[None, :]
    cross = lax.dot_general(qm.astype(jnp.bfloat16), xm.astype(jnp.bfloat16),
                            (((1,), (1,)), ((), ())),
                            preferred_element_type=jnp.float32)
    d2 = qn + xn - 2.0 * cross

    SENT = 32768.0
    iota_n = lax.broadcasted_iota(jnp.int32, (MB, N), 1).astype(jnp.float32)
    key = jnp.where(d2 < RADIUS2, iota_n, SENT)
    iota32 = lax.broadcasted_iota(jnp.int32, (1, NS), 1)
    idxf = jnp.zeros((MB, NS), jnp.float32)
    cur = jnp.full((MB, 1), -1.0, jnp.float32)
    for s in range(NS):
        cur = jnp.min(jnp.where(key > cur, key, SENT),
                      axis=1, keepdims=True)
        idxf = idxf + cur * (iota32 == s).astype(jnp.float32)
    valid = (idxf < SENT - 0.5).astype(jnp.float32)

    hi = jnp.floor(idxf * (1.0 / 64.0))
    lo = idxf - hi * 64.0
    ih = lax.broadcasted_iota(jnp.int32, (MB, NS, 128), 2).astype(jnp.float32)
    oh = (hi[:, :, None] == ih).astype(jnp.float32)
    il = lax.broadcasted_iota(jnp.int32, (MB, NS, 64), 2).astype(jnp.float32)
    ol = (lo[:, :, None] == il).astype(jnp.float32)

    g1 = lax.dot_general(oh, xr_ref[0], (((2,), (0,)), ((), ())),
                         precision=lax.Precision.HIGHEST,
                         preferred_element_type=jnp.float32)
    oltile = jnp.concatenate([ol, ol, ol], axis=2)
    g2 = g1 * oltile
    rel192 = g2 - qrep_ref[0][:, None, :] * oltile
    h1p = lax.dot_general(rel192.astype(jnp.bfloat16),
                          wb_ref[...].astype(jnp.bfloat16),
                          (((2,), (0,)), ((), ())),
                          preferred_element_type=jnp.float32)
    h1 = jnp.maximum(h1p + b1_ref[0][None, None, :], 0.0)
    h2p = lax.dot_general(h1.astype(jnp.bfloat16),
                          w2_ref[...].astype(jnp.bfloat16),
                          (((2,), (1,)), ((), ())),
                          preferred_element_type=jnp.float32)
    h2 = jnp.maximum(h2p + b2_ref[0][None, None, :], 0.0)
    pooled = jnp.max(h2 * valid[:, :, None], axis=1)
    out_ref[...] = pooled[None]


def kernel(xyz, new_xyz, W1, b1, W2, b2):
    B = xyz.shape[0]
    xr = xyz.reshape(B, 128, 64, 3).transpose(0, 1, 3, 2).reshape(B, 128, 192)
    qrep = jnp.repeat(new_xyz, 64, axis=2)
    wb = jnp.repeat(W1.T, 64, axis=0)
    b1r = b1.reshape(1, 32)
    b2r = b2.reshape(1, 32)

    grid = (B, M // MB)
    out = pl.pallas_call(
        _kernel,
        grid=grid,
        in_specs=[
            pl.BlockSpec((1, MB, 3), lambda b, m: (b, m, 0)),
            pl.BlockSpec((1, N, 3), lambda b, m: (b, 0, 0)),
            pl.BlockSpec((1, 128, 192), lambda b, m: (b, 0, 0)),
            pl.BlockSpec((1, MB, 192), lambda b, m: (b, m, 0)),
            pl.BlockSpec((192, 32), lambda b, m: (0, 0)),
            pl.BlockSpec((32, 3), lambda b, m: (0, 0)),
            pl.BlockSpec((1, 32), lambda b, m: (0, 0)),
            pl.BlockSpec((32, 32), lambda b, m: (0, 0)),
            pl.BlockSpec((1, 32), lambda b, m: (0, 0)),
        ],
        out_specs=pl.BlockSpec((1, MB, 32), lambda b, m: (b, m, 0)),
        out_shape=jax.ShapeDtypeStruct((B, M, 32), jnp.float32),
    )(new_xyz, xyz, xr, qrep, wb, W1, b1r, W2, b2r)
    feats = out.transpose(0, 2, 1)
    return new_xyz, feats


# speedup vs baseline: 1.4005x; 1.1524x over previous
import jax
import jax.numpy as jnp
from jax import lax
from jax.experimental import pallas as pl

RADIUS2 = 0.1 * 0.1
NS = 32
N = 8192
M = 2048
MB = 512
BIG = 1.0e9


def _kernel(qm_ref, xm_ref, xr0_ref, xr1_ref, xr2_ref, qrep_ref, wb_ref,
            w1_ref, b1_ref, w2_ref, b2_ref, out_ref):
    qm = qm_ref[0]
    xm = xm_ref[0]
    qn = jnp.sum(qm * qm, axis=1)[:, None]
    xn = jnp.sum(xm * xm, axis=1)[None, :]
    cross = lax.dot_general(qm.astype(jnp.bfloat16), xm.astype(jnp.bfloat16),
                            (((1,), (1,)), ((), ())),
                            preferred_element_type=jnp.float32)
    d2 = qn + xn - 2.0 * cross

    SENT = 32768.0
    iota_n = lax.broadcasted_iota(jnp.int32, (MB, N), 1).astype(jnp.float32)
    key = jnp.where(d2 < RADIUS2, iota_n, SENT)
    iota32 = lax.broadcasted_iota(jnp.int32, (1, NS), 1)
    idxf = jnp.zeros((MB, NS), jnp.float32)
    cur = jnp.full((MB, 1), -1.0, jnp.float32)
    for s in range(NS):
        cur = jnp.min(jnp.where(key > cur, key, SENT),
                      axis=1, keepdims=True)
        idxf = idxf + cur * (iota32 == s).astype(jnp.float32)
    valid = (idxf < SENT - 0.5).astype(jnp.float32)

    hi = jnp.floor(idxf * (1.0 / 64.0))
    lo = idxf - hi * 64.0
    ih = lax.broadcasted_iota(jnp.int32, (MB, NS, 128), 2).astype(jnp.float32)
    oh = (hi[:, :, None] == ih).astype(jnp.float32)
    il = lax.broadcasted_iota(jnp.int32, (MB, NS, 64), 2).astype(jnp.float32)
    ol = (lo[:, :, None] == il).astype(jnp.float32)

    ohb = oh.astype(jnp.bfloat16)
    g1 = (lax.dot_general(ohb, xr0_ref[0], (((2,), (0,)), ((), ())),
                          preferred_element_type=jnp.float32)
          + lax.dot_general(ohb, xr1_ref[0], (((2,), (0,)), ((), ())),
                            preferred_element_type=jnp.float32)
          + lax.dot_general(ohb, xr2_ref[0], (((2,), (0,)), ((), ())),
                            preferred_element_type=jnp.float32))
    oltile = jnp.concatenate([ol, ol, ol], axis=2)
    g2 = g1 * oltile
    rel192 = g2 - qrep_ref[0][:, None, :] * oltile
    h1p = lax.dot_general(rel192.astype(jnp.bfloat16),
                          wb_ref[...].astype(jnp.bfloat16),
                          (((2,), (0,)), ((), ())),
                          preferred_element_type=jnp.float32)
    h1 = jnp.maximum(h1p + b1_ref[0][None, None, :], 0.0)
    h2p = lax.dot_general(h1.astype(jnp.bfloat16),
                          w2_ref[...].astype(jnp.bfloat16),
                          (((2,), (1,)), ((), ())),
                          preferred_element_type=jnp.float32)
    h2 = jnp.maximum(h2p + b2_ref[0][None, None, :], 0.0)
    pooled = jnp.max(h2 * valid[:, :, None], axis=1)
    out_ref[...] = pooled[None]


def kernel(xyz, new_xyz, W1, b1, W2, b2):
    B = xyz.shape[0]
    xr = xyz.reshape(B, 128, 64, 3).transpose(0, 1, 3, 2).reshape(B, 128, 192)
    xr0 = xr.astype(jnp.bfloat16)
    r1 = xr - xr0.astype(jnp.float32)
    xr1 = r1.astype(jnp.bfloat16)
    xr2 = (r1 - xr1.astype(jnp.float32)).astype(jnp.bfloat16)
    qrep = jnp.repeat(new_xyz, 64, axis=2)
    wb = jnp.repeat(W1.T, 64, axis=0)
    b1r = b1.reshape(1, 32)
    b2r = b2.reshape(1, 32)

    grid = (B, M // MB)
    out = pl.pallas_call(
        _kernel,
        grid=grid,
        in_specs=[
            pl.BlockSpec((1, MB, 3), lambda b, m: (b, m, 0)),
            pl.BlockSpec((1, N, 3), lambda b, m: (b, 0, 0)),
            pl.BlockSpec((1, 128, 192), lambda b, m: (b, 0, 0)),
            pl.BlockSpec((1, 128, 192), lambda b, m: (b, 0, 0)),
            pl.BlockSpec((1, 128, 192), lambda b, m: (b, 0, 0)),
            pl.BlockSpec((1, MB, 192), lambda b, m: (b, m, 0)),
            pl.BlockSpec((192, 32), lambda b, m: (0, 0)),
            pl.BlockSpec((32, 3), lambda b, m: (0, 0)),
            pl.BlockSpec((1, 32), lambda b, m: (0, 0)),
            pl.BlockSpec((32, 32), lambda b, m: (0, 0)),
            pl.BlockSpec((1, 32), lambda b, m: (0, 0)),
        ],
        out_specs=pl.BlockSpec((1, MB, 32), lambda b, m: (b, m, 0)),
        out_shape=jax.ShapeDtypeStruct((B, M, 32), jnp.float32),
    )(new_xyz, xyz, xr0, xr1, xr2, qrep, wb, W1, b1r, W2, b2r)
    feats = out.transpose(0, 2, 1)
    return new_xyz, feats


# speedup vs baseline: 1.4834x; 1.0592x over previous
import jax
import jax.numpy as jnp
from jax import lax
from jax.experimental import pallas as pl

RADIUS2 = 0.1 * 0.1
NS = 32
N = 8192
M = 2048
MB = 1024
BIG = 1.0e9


def _kernel(qm_ref, xm_ref, xr0_ref, xr1_ref, xr2_ref, qrep_ref, wb_ref,
            w1_ref, b1_ref, w2_ref, b2_ref, out_ref):
    qm = qm_ref[0]
    xm = xm_ref[0]
    qn = jnp.sum(qm * qm, axis=1)[:, None]
    xn = jnp.sum(xm * xm, axis=1)[None, :]
    cross = lax.dot_general(qm.astype(jnp.bfloat16), xm.astype(jnp.bfloat16),
                            (((1,), (1,)), ((), ())),
                            preferred_element_type=jnp.float32)
    d2 = qn + xn - 2.0 * cross

    SENT = 32768.0
    iota_n = lax.broadcasted_iota(jnp.int32, (MB, N), 1).astype(jnp.float32)
    key = jnp.where(d2 < RADIUS2, iota_n, SENT)
    iota32 = lax.broadcasted_iota(jnp.int32, (1, NS), 1)
    idxf = jnp.zeros((MB, NS), jnp.float32)
    cur = jnp.full((MB, 1), -1.0, jnp.float32)
    for s in range(NS):
        cur = jnp.min(jnp.where(key > cur, key, SENT),
                      axis=1, keepdims=True)
        idxf = idxf + cur * (iota32 == s).astype(jnp.float32)
    valid = (idxf < SENT - 0.5).astype(jnp.float32)

    hi = jnp.floor(idxf * (1.0 / 64.0))
    lo = idxf - hi * 64.0
    ih = lax.broadcasted_iota(jnp.int32, (MB, NS, 128), 2).astype(jnp.float32)
    oh = (hi[:, :, None] == ih).astype(jnp.float32)
    il = lax.broadcasted_iota(jnp.int32, (MB, NS, 64), 2).astype(jnp.float32)
    ol = (lo[:, :, None] == il).astype(jnp.float32)

    ohb = oh.astype(jnp.bfloat16)
    g1 = (lax.dot_general(ohb, xr0_ref[0], (((2,), (0,)), ((), ())),
                          preferred_element_type=jnp.float32)
          + lax.dot_general(ohb, xr1_ref[0], (((2,), (0,)), ((), ())),
                            preferred_element_type=jnp.float32)
          + lax.dot_general(ohb, xr2_ref[0], (((2,), (0,)), ((), ())),
                            preferred_element_type=jnp.float32))
    oltile = jnp.concatenate([ol, ol, ol], axis=2)
    g2 = g1 * oltile
    rel192 = g2 - qrep_ref[0][:, None, :] * oltile
    h1p = lax.dot_general(rel192.astype(jnp.bfloat16),
                          wb_ref[...].astype(jnp.bfloat16),
                          (((2,), (0,)), ((), ())),
                          preferred_element_type=jnp.float32)
    h1 = jnp.maximum(h1p + b1_ref[0][None, None, :], 0.0)
    h2p = lax.dot_general(h1.astype(jnp.bfloat16),
                          w2_ref[...].astype(jnp.bfloat16),
                          (((2,), (1,)), ((), ())),
                          preferred_element_type=jnp.float32)
    h2 = jnp.maximum(h2p + b2_ref[0][None, None, :], 0.0)
    pooled = jnp.max(h2 * valid[:, :, None], axis=1)
    out_ref[...] = pooled[None]


def kernel(xyz, new_xyz, W1, b1, W2, b2):
    B = xyz.shape[0]
    xr = xyz.reshape(B, 128, 64, 3).transpose(0, 1, 3, 2).reshape(B, 128, 192)
    xr0 = xr.astype(jnp.bfloat16)
    r1 = xr - xr0.astype(jnp.float32)
    xr1 = r1.astype(jnp.bfloat16)
    xr2 = (r1 - xr1.astype(jnp.float32)).astype(jnp.bfloat16)
    qrep = jnp.repeat(new_xyz, 64, axis=2)
    wb = jnp.repeat(W1.T, 64, axis=0)
    b1r = b1.reshape(1, 32)
    b2r = b2.reshape(1, 32)

    grid = (B, M // MB)
    out = pl.pallas_call(
        _kernel,
        grid=grid,
        in_specs=[
            pl.BlockSpec((1, MB, 3), lambda b, m: (b, m, 0)),
            pl.BlockSpec((1, N, 3), lambda b, m: (b, 0, 0)),
            pl.BlockSpec((1, 128, 192), lambda b, m: (b, 0, 0)),
            pl.BlockSpec((1, 128, 192), lambda b, m: (b, 0, 0)),
            pl.BlockSpec((1, 128, 192), lambda b, m: (b, 0, 0)),
            pl.BlockSpec((1, MB, 192), lambda b, m: (b, m, 0)),
            pl.BlockSpec((192, 32), lambda b, m: (0, 0)),
            pl.BlockSpec((32, 3), lambda b, m: (0, 0)),
            pl.BlockSpec((1, 32), lambda b, m: (0, 0)),
            pl.BlockSpec((32, 32), lambda b, m: (0, 0)),
            pl.BlockSpec((1, 32), lambda b, m: (0, 0)),
        ],
        out_specs=pl.BlockSpec((1, MB, 32), lambda b, m: (b, m, 0)),
        out_shape=jax.ShapeDtypeStruct((B, M, 32), jnp.float32),
    )(new_xyz, xyz, xr0, xr1, xr2, qrep, wb, W1, b1r, W2, b2r)
    feats = out.transpose(0, 2, 1)
    return new_xyz, feats
